# Initial kernel scaffold; baseline (speedup 1.0000x reference)
import jax
import jax.numpy as jnp
from jax.experimental import pallas as pl


def kernel(x, table):
    raise NotImplementedError("write your pallas kernel here")


# speedup vs baseline: 2.5653x; 2.5653x over previous
import jax
import jax.numpy as jnp
from jax import lax
from jax.experimental import pallas as pl
from jax.experimental.pallas import tpu as pltpu
from jax.experimental.pallas import tpu_sc as plsc

BATCH = 16384
HIST_LEN = 200
FEATURES = 64
N = BATCH * HIST_LEN

NUM_CORES = 2
NUM_SUBCORES = 16
NW = NUM_CORES * NUM_SUBCORES
PER_W = N // NW

GROUP = 4
QROW = GROUP * FEATURES
CHUNK = 512
QPC = CHUNK // GROUP
ITERS = PER_W // CHUNK
L = 16


def _body(x_hbm, qtab_hbm, out_hbm, x_v, idxg_v, rows_v, sem_g):
    wid = lax.axis_index("s") * NUM_CORES + lax.axis_index("c")

    def step(i, carry):
        base = pl.multiple_of(wid * PER_W + i * CHUNK, CHUNK)
        pltpu.sync_copy(x_hbm.at[pl.ds(base, CHUNK)], x_v)
        lane = lax.iota(jnp.int32, L)
        for j in range(QPC // L):
            p = (j * L + lane) * GROUP
            q = plsc.load_gather(x_v, [p])
            for r in range(1, GROUP):
                q = q * 2 + plsc.load_gather(x_v, [p + r])
            idxg_v[pl.ds(j * L, L)] = q
        pltpu.async_copy(qtab_hbm.at[idxg_v], rows_v, sem_g).wait()
        obase = pl.multiple_of(base // GROUP, QPC)
        pltpu.sync_copy(rows_v, out_hbm.at[pl.ds(obase, QPC)])
        return carry

    lax.fori_loop(0, ITERS, step, 0)


@jax.jit
def _lookup(x_flat, qtab):
    mesh = plsc.VectorSubcoreMesh(core_axis_name="c", subcore_axis_name="s")
    f = pl.kernel(
        _body,
        out_type=jax.ShapeDtypeStruct((N // GROUP, QROW), jnp.float32),
        mesh=mesh,
        scratch_types=[
            pltpu.VMEM((CHUNK,), jnp.int32),
            pltpu.VMEM((QPC,), jnp.int32),
            pltpu.VMEM((QPC, QROW), jnp.float32),
            pltpu.SemaphoreType.DMA,
        ],
        compiler_params=pltpu.CompilerParams(needs_layout_passes=False),
    )
    return f(x_flat, qtab)


def kernel(x, table):
    q = jnp.arange(16, dtype=jnp.int32)
    bits = jnp.stack([(q >> 3) & 1, (q >> 2) & 1, (q >> 1) & 1, q & 1], axis=1)
    qtab = table[bits].reshape(16, QROW)
    out = _lookup(x.reshape(N), qtab)
    return out.reshape(BATCH, HIST_LEN, FEATURES)


# speedup vs baseline: 2.5868x; 1.0084x over previous
import jax
import jax.numpy as jnp
from jax import lax
from jax.experimental import pallas as pl
from jax.experimental.pallas import tpu as pltpu
from jax.experimental.pallas import tpu_sc as plsc

BATCH = 16384
HIST_LEN = 200
FEATURES = 64
N = BATCH * HIST_LEN

NUM_CORES = 2
NUM_SUBCORES = 16
NW = NUM_CORES * NUM_SUBCORES
PER_W = N // NW

GROUP = 4
QROW = GROUP * FEATURES
CHUNK = 512
QPC = CHUNK // GROUP
ITERS = PER_W // CHUNK
L = 16
NB = 3


def _body(x_hbm, qtab_hbm, out_hbm, x_v, idxg_v, rows_v, sem_x, sem_g, sem_o):
    wid = lax.axis_index("s") * NUM_CORES + lax.axis_index("c")
    lane = lax.iota(jnp.int32, L)

    def x_copy(i, bx):
        base = pl.multiple_of(wid * PER_W + i * CHUNK, CHUNK)
        return pltpu.make_async_copy(x_hbm.at[pl.ds(base, CHUNK)], x_v.at[bx],
                                     sem_x)

    def rows_slice(b):
        return rows_v.at[pl.ds(pl.multiple_of(b * QPC, QPC), QPC)]

    def gather(b):
        return pltpu.make_async_copy(qtab_hbm.at[idxg_v.at[b]], rows_slice(b),
                                     sem_g)

    def write(i, b):
        obase = pl.multiple_of((wid * PER_W + i * CHUNK) // GROUP, QPC)
        return pltpu.make_async_copy(rows_slice(b),
                                     out_hbm.at[pl.ds(obase, QPC)], sem_o)

    x_copy(0, 0).start()

    def step(i, carry):
        b = lax.rem(i, NB)
        bx = lax.rem(i, 2)
        bp = lax.rem(i + (NB - 1), NB)

        x_copy(i, bx).wait()
        bx_vec = jnp.full((L,), bx, jnp.int32)
        for j in range(QPC // L):
            p = (j * L + lane) * GROUP
            q = plsc.load_gather(x_v, [bx_vec, p])
            for r in range(1, GROUP):
                q = q * 2 + plsc.load_gather(x_v, [bx_vec, p + r])
            idxg_v[b, pl.ds(j * L, L)] = q

        @pl.when(i >= NB)
        def _():
            write(i - NB, b).wait()

        @pl.when(i + 1 < ITERS)
        def _():
            x_copy(i + 1, 1 - bx).start()

        gather(b).start()

        @pl.when(i >= 1)
        def _():
            gather(bp).wait()
            write(i - 1, bp).start()

        return carry

    lax.fori_loop(0, ITERS, step, 0)

    bl = (ITERS - 1) % NB
    gather(bl).wait()
    write(ITERS - 1, bl).start()
    for k in range(NB):
        write(ITERS - 1, bl).wait()


@jax.jit
def _lookup(x_flat, qtab):
    mesh = plsc.VectorSubcoreMesh(core_axis_name="c", subcore_axis_name="s")
    f = pl.kernel(
        _body,
        out_type=jax.ShapeDtypeStruct((N // GROUP, QROW), jnp.float32),
        mesh=mesh,
        scratch_types=[
            pltpu.VMEM((2, CHUNK), jnp.int32),
            pltpu.VMEM((NB, QPC), jnp.int32),
            pltpu.VMEM((NB * QPC, QROW), jnp.float32),
            pltpu.SemaphoreType.DMA,
            pltpu.SemaphoreType.DMA,
            pltpu.SemaphoreType.DMA,
        ],
        compiler_params=pltpu.CompilerParams(needs_layout_passes=False),
    )
    return f(x_flat, qtab)


def kernel(x, table):
    q = jnp.arange(16, dtype=jnp.int32)
    bits = jnp.stack([(q >> 3) & 1, (q >> 2) & 1, (q >> 1) & 1, q & 1], axis=1)
    qtab = table[bits].reshape(16, QROW)
    out = _lookup(x.reshape(N), qtab)
    return out.reshape(BATCH, HIST_LEN, FEATURES)


# speedup vs baseline: 4.6291x; 1.7895x over previous
import jax
import jax.numpy as jnp
from jax import lax
from jax.experimental import pallas as pl
from jax.experimental.pallas import tpu as pltpu
from jax.experimental.pallas import tpu_sc as plsc

BATCH = 16384
HIST_LEN = 200
FEATURES = 64
N = BATCH * HIST_LEN

NUM_CORES = 2
NUM_SUBCORES = 16
NW = NUM_CORES * NUM_SUBCORES
PER_W = N // NW

GROUP = 8
QROW = GROUP * FEATURES
CHUNK = 512
QPC = CHUNK // GROUP
ITERS = PER_W // CHUNK
L = 16
NB = 3


def _body(x_hbm, qtab_hbm, out_hbm, x_v, idxg_v, rows_v, sem_x, sem_g, sem_o):
    wid = lax.axis_index("s") * NUM_CORES + lax.axis_index("c")
    lane = lax.iota(jnp.int32, L)

    def x_copy(i, bx):
        base = pl.multiple_of(wid * PER_W + i * CHUNK, CHUNK)
        return pltpu.make_async_copy(x_hbm.at[pl.ds(base, CHUNK)], x_v.at[bx],
                                     sem_x)

    def rows_slice(b):
        return rows_v.at[pl.ds(pl.multiple_of(b * QPC, QPC), QPC)]

    def gather(b):
        return pltpu.make_async_copy(qtab_hbm.at[idxg_v.at[b]], rows_slice(b),
                                     sem_g)

    def write(i, b):
        obase = pl.multiple_of((wid * PER_W + i * CHUNK) // GROUP, QPC)
        return pltpu.make_async_copy(rows_slice(b),
                                     out_hbm.at[pl.ds(obase, QPC)], sem_o)

    x_copy(0, 0).start()

    def step(i, carry):
        b = lax.rem(i, NB)
        bx = lax.rem(i, 2)
        bp = lax.rem(i + (NB - 1), NB)

        x_copy(i, bx).wait()
        bx_vec = jnp.full((L,), bx, jnp.int32)
        for j in range(QPC // L):
            p = (j * L + lane) * GROUP
            q = plsc.load_gather(x_v, [bx_vec, p])
            for r in range(1, GROUP):
                q = q * 2 + plsc.load_gather(x_v, [bx_vec, p + r])
            idxg_v[b, pl.ds(j * L, L)] = q

        @pl.when(i >= NB)
        def _():
            write(i - NB, b).wait()

        @pl.when(i + 1 < ITERS)
        def _():
            x_copy(i + 1, 1 - bx).start()

        gather(b).start()

        @pl.when(i >= 1)
        def _():
            gather(bp).wait()
            write(i - 1, bp).start()

        return carry

    lax.fori_loop(0, ITERS, step, 0)

    bl = (ITERS - 1) % NB
    gather(bl).wait()
    write(ITERS - 1, bl).start()
    for k in range(NB):
        write(ITERS - 1, bl).wait()


@jax.jit
def _lookup(x_flat, qtab):
    mesh = plsc.VectorSubcoreMesh(core_axis_name="c", subcore_axis_name="s")
    f = pl.kernel(
        _body,
        out_type=jax.ShapeDtypeStruct((N // GROUP, QROW), jnp.float32),
        mesh=mesh,
        scratch_types=[
            pltpu.VMEM((2, CHUNK), jnp.int32),
            pltpu.VMEM((NB, QPC), jnp.int32),
            pltpu.VMEM((NB * QPC, QROW), jnp.float32),
            pltpu.SemaphoreType.DMA,
            pltpu.SemaphoreType.DMA,
            pltpu.SemaphoreType.DMA,
        ],
        compiler_params=pltpu.CompilerParams(needs_layout_passes=False),
    )
    return f(x_flat, qtab)


def kernel(x, table):
    q = jnp.arange(1 << GROUP, dtype=jnp.int32)
    bits = jnp.stack([(q >> (GROUP - 1 - r)) & 1 for r in range(GROUP)], axis=1)
    qtab = table[bits].reshape(1 << GROUP, QROW)
    out = _lookup(x.reshape(N), qtab)
    return out.reshape(BATCH, HIST_LEN, FEATURES)


# speedup vs baseline: 5.9038x; 1.2754x over previous
import jax
import jax.numpy as jnp
from jax import lax
from jax.experimental import pallas as pl
from jax.experimental.pallas import tpu as pltpu
from jax.experimental.pallas import tpu_sc as plsc

BATCH = 16384
HIST_LEN = 200
FEATURES = 64
N = BATCH * HIST_LEN

NUM_CORES = 2
NUM_SUBCORES = 16
NW = NUM_CORES * NUM_SUBCORES
PER_W = N // NW

CHUNK = 512
ITERS = PER_W // CHUNK
L = 16
NQ = FEATURES // L
NB = 3
RB = 16


def _body(x_hbm, tab_hbm, out_hbm, x_v, tab_v, rows_v, sem_x, sem_o):
    wid = lax.axis_index("s") * NUM_CORES + lax.axis_index("c")

    def x_copy(i, bx):
        base = pl.multiple_of(wid * PER_W + i * CHUNK, CHUNK)
        return pltpu.make_async_copy(x_hbm.at[pl.ds(base, CHUNK)], x_v.at[bx],
                                     sem_x)

    CW = CHUNK * FEATURES

    def rows_slice(b):
        return rows_v.at[pl.ds(pl.multiple_of(b * CW, CW), CW)]

    def write(i, b):
        obase = pl.multiple_of((wid * PER_W + i * CHUNK) * FEATURES, CW)
        return pltpu.make_async_copy(rows_slice(b),
                                     out_hbm.at[pl.ds(obase, CW)], sem_o)

    pltpu.sync_copy(tab_hbm, tab_v)
    w0 = [tab_v[0, pl.ds(q * L, L)] for q in range(NQ)]
    w1 = [tab_v[1, pl.ds(q * L, L)] for q in range(NQ)]
    one = jnp.full((L,), 1, jnp.int32)

    x_copy(0, 0).start()

    def step(i, carry):
        b = lax.rem(i, NB)
        bx = lax.rem(i, 2)

        x_copy(i, bx).wait()

        @pl.when(i + 1 < ITERS)
        def _():
            x_copy(i + 1, 1 - bx).start()

        @pl.when(i >= NB)
        def _():
            write(i - NB, b).wait()

        rbase0 = b * CW

        def block(j, carry2):
            rbase = rbase0 + j * RB * FEATURES
            xv = x_v[bx, pl.ds(j * RB, L)]
            for t in range(RB):
                m = jnp.full((L,), xv[t], jnp.int32) == one
                for q in range(NQ):
                    off = pl.multiple_of(rbase + t * FEATURES + q * L, L)
                    rows_v[pl.ds(off, L)] = jnp.where(m, w1[q], w0[q])
            return carry2

        lax.fori_loop(0, CHUNK // RB, block, 0)

        write(i, b).start()
        return carry

    lax.fori_loop(0, ITERS, step, 0)

    for k in range(NB):
        write(ITERS - 1, (ITERS - 1) % NB).wait()


@jax.jit
def _lookup(x_flat, table):
    f = pl.kernel(
        _body,
        out_type=jax.ShapeDtypeStruct((N * FEATURES,), jnp.float32),
        mesh=plsc.VectorSubcoreMesh(core_axis_name="c", subcore_axis_name="s"),
        scratch_types=[
            pltpu.VMEM((2, CHUNK), jnp.int32),
            pltpu.VMEM((2, FEATURES), jnp.float32),
            pltpu.VMEM((NB * CHUNK * FEATURES,), jnp.float32),
            pltpu.SemaphoreType.DMA,
            pltpu.SemaphoreType.DMA,
        ],
        compiler_params=pltpu.CompilerParams(needs_layout_passes=False),
    )
    return f(x_flat, table)


def kernel(x, table):
    out = _lookup(x.reshape(N), table)
    return out.reshape(BATCH, HIST_LEN, FEATURES)
